# baseline (device time: 401408 ns/iter reference)
import jax
import jax.numpy as jnp
from jax import lax
from jax.experimental import pallas as pl
from jax.experimental.pallas import tpu as pltpu

N_DEV = 16
MC = 256
KS = 256
N = 8192
NH = N // 2

_MESH = pl.DeviceIdType.MESH


def _mesh_from_ring(p):
    p = lax.rem(p + 2 * N_DEV, N_DEV)
    w = p // 4
    zp = p % 4
    par = w % 2
    z = zp * (1 - 2 * par) + 3 * par
    return 4 * z + w


def _ring_from_mesh(m):
    w = m % 4
    z = m // 4
    par = w % 2
    zz = z * (1 - 2 * par) + 3 * par
    return 4 * w + zz


def kernel(x, w_mat):
    x = x.astype(jnp.bfloat16)
    w_mat = w_mat.astype(jnp.bfloat16)

    def body(x_ref, w_ref, out_ref,
             acc_cw, acc_ccw, rcv_cw, rcv_ccw,
             send_cw, recv_cw, send_ccw, recv_ccw,
             credit_cw, credit_ccw,
             amax_src, amax_buf, amax_send, amax_recv):
        my_m = lax.axis_index("i")
        my_p = _ring_from_mesh(my_m)
        right_m = _mesh_from_ring(my_p + 1)
        left_m = _mesh_from_ring(my_p - 1)

        barrier = pltpu.get_barrier_semaphore()
        for nbr in (left_m, right_m):
            pl.semaphore_signal(barrier, inc=1, device_id=(nbr,),
                                device_id_type=_MESH)
        pl.semaphore_wait(barrier, 2)

        pl.semaphore_signal(credit_cw, inc=1, device_id=(left_m,),
                            device_id_type=_MESH)
        pl.semaphore_signal(credit_ccw, inc=1, device_id=(right_m,),
                            device_id_type=_MESH)

        def partial_chunk(c, col0):
            xa = x_ref[pl.ds(c * MC, MC), :]
            wb = w_ref[:, col0:col0 + NH]
            return lax.dot_general(
                xa, wb, dimension_numbers=(((1,), (0,)), ((), ())),
                preferred_element_type=jnp.float32)

        acc_cw[...] = partial_chunk(_mesh_from_ring(my_p - 1), 0
                                    ).astype(jnp.bfloat16)
        acc_ccw[...] = partial_chunk(_mesh_from_ring(my_p + 1), NH
                                     ).astype(jnp.bfloat16)

        y1 = None
        y2 = None
        for s in range(N_DEV - 1):
            pl.semaphore_wait(credit_cw, 1)
            cw = pltpu.make_async_remote_copy(
                src_ref=acc_cw, dst_ref=rcv_cw,
                send_sem=send_cw, recv_sem=recv_cw,
                device_id=(right_m,), device_id_type=_MESH)
            cw.start()
            pl.semaphore_wait(credit_ccw, 1)
            ccw = pltpu.make_async_remote_copy(
                src_ref=acc_ccw, dst_ref=rcv_ccw,
                send_sem=send_ccw, recv_sem=recv_ccw,
                device_id=(left_m,), device_id_type=_MESH)
            ccw.start()

            if s < N_DEV - 2:
                p1 = partial_chunk(_mesh_from_ring(my_p - 2 - s), 0)
                p2 = partial_chunk(_mesh_from_ring(my_p + 2 + s), NH)
                cw.wait_recv()
                cw.wait_send()
                acc_cw[...] = (rcv_cw[...].astype(jnp.float32) + p1
                               ).astype(jnp.bfloat16)
                pl.semaphore_signal(credit_cw, inc=1, device_id=(left_m,),
                                    device_id_type=_MESH)
                ccw.wait_recv()
                ccw.wait_send()
                acc_ccw[...] = (rcv_ccw[...].astype(jnp.float32) + p2
                                ).astype(jnp.bfloat16)
                pl.semaphore_signal(credit_ccw, inc=1, device_id=(right_m,),
                                    device_id_type=_MESH)
            else:
                p1 = partial_chunk(my_m, 0)
                p2 = partial_chunk(my_m, NH)
                cw.wait_recv()
                cw.wait_send()
                y1 = jnp.maximum(rcv_cw[...].astype(jnp.float32) + p1, 0.0)
                ccw.wait_recv()
                ccw.wait_send()
                y2 = jnp.maximum(rcv_ccw[...].astype(jnp.float32) + p2, 0.0)

        out_ref[:, :NH] = y1
        out_ref[:, NH:] = y2
        local_amax = jnp.maximum(jnp.max(y1), jnp.max(y2))

        amax_src[...] = jnp.broadcast_to(local_amax, (1, 128)
                                         ).astype(jnp.float32)
        descs = []
        for r in range(1, N_DEV):
            d = pltpu.make_async_remote_copy(
                src_ref=amax_src, dst_ref=amax_buf.at[pl.ds(r, 1)],
                send_sem=amax_send.at[r], recv_sem=amax_recv.at[r],
                device_id=(_mesh_from_ring(my_p + r),),
                device_id_type=_MESH)
            d.start()
            descs.append(d)
        for d in descs:
            d.wait_send()
            d.wait_recv()
        gmax = jnp.maximum(jnp.max(amax_buf[pl.ds(1, N_DEV - 1), 0]),
                           local_amax)

        scale = gmax / 127.0
        inv = 127.0 / gmax
        out_ref[:, :NH] = jnp.clip(jnp.round(out_ref[:, :NH] * inv),
                                   0.0, 127.0) * scale
        out_ref[:, NH:] = jnp.clip(jnp.round(out_ref[:, NH:] * inv),
                                   0.0, 127.0) * scale

    return pl.pallas_call(
        body,
        out_shape=jax.ShapeDtypeStruct((MC, N), jnp.float32),
        in_specs=[pl.BlockSpec(memory_space=pltpu.VMEM),
                  pl.BlockSpec(memory_space=pltpu.VMEM)],
        out_specs=pl.BlockSpec(memory_space=pltpu.VMEM),
        scratch_shapes=[
            pltpu.VMEM((MC, NH), jnp.bfloat16),
            pltpu.VMEM((MC, NH), jnp.bfloat16),
            pltpu.VMEM((MC, NH), jnp.bfloat16),
            pltpu.VMEM((MC, NH), jnp.bfloat16),
            pltpu.SemaphoreType.DMA,
            pltpu.SemaphoreType.DMA,
            pltpu.SemaphoreType.DMA,
            pltpu.SemaphoreType.DMA,
            pltpu.SemaphoreType.REGULAR,
            pltpu.SemaphoreType.REGULAR,
            pltpu.VMEM((1, 128), jnp.float32),
            pltpu.VMEM((N_DEV, 128), jnp.float32),
            pltpu.SemaphoreType.DMA((N_DEV,)),
            pltpu.SemaphoreType.DMA((N_DEV,)),
        ],
        compiler_params=pltpu.CompilerParams(collective_id=0),
    )(x, w_mat)


# device time: 362164 ns/iter; 1.1084x vs baseline; 1.1084x over previous
import jax
import jax.numpy as jnp
from jax import lax
from jax.experimental import pallas as pl
from jax.experimental.pallas import tpu as pltpu

N_DEV = 16
MC = 256
KS = 256
N = 8192
NH = N // 2
NQ = NH // 2

_MESH = pl.DeviceIdType.MESH


def _mesh_from_ring(p):
    p = lax.rem(p + 2 * N_DEV, N_DEV)
    w = p // 4
    zp = p % 4
    par = w % 2
    z = zp * (1 - 2 * par) + 3 * par
    return 4 * z + w


def _ring_from_mesh(m):
    w = m % 4
    z = m // 4
    par = w % 2
    zz = z * (1 - 2 * par) + 3 * par
    return 4 * w + zz


def kernel(x, w_mat):
    x = x.astype(jnp.bfloat16)
    w_mat = w_mat.astype(jnp.bfloat16)

    def body(x_ref, w_ref, out_ref,
             acc_cw0, acc_cw1, acc_ccw0, acc_ccw1,
             rcv_cw0, rcv_cw1, rcv_ccw0, rcv_ccw1,
             send_sems, recv_sems,
             credits,
             amax_src, amax_buf, amax_send, amax_recv):
        my_m = lax.axis_index("i")
        my_p = _ring_from_mesh(my_m)
        right_m = _mesh_from_ring(my_p + 1)
        left_m = _mesh_from_ring(my_p - 1)

        barrier = pltpu.get_barrier_semaphore()
        for nbr in (left_m, right_m):
            pl.semaphore_signal(barrier, inc=1, device_id=(nbr,),
                                device_id_type=_MESH)
        pl.semaphore_wait(barrier, 2)

        def partial_chunk(c, col0):
            xa = x_ref[pl.ds(c * MC, MC), :]
            wb = w_ref[:, col0:col0 + NQ]
            return lax.dot_general(
                xa, wb, dimension_numbers=(((1,), (0,)), ((), ())),
                preferred_element_type=jnp.float32)

        lanes = (
            (0, acc_cw0, rcv_cw0, right_m, left_m, 0, +1),
            (1, acc_ccw0, rcv_ccw0, left_m, right_m, NH, -1),
            (2, acc_cw1, rcv_cw1, right_m, left_m, NQ, +1),
            (3, acc_ccw1, rcv_ccw1, left_m, right_m, NH + NQ, -1),
        )

        for li, _, _, _, credit_to, _, _ in lanes:
            pl.semaphore_signal(credits.at[li], inc=1,
                                device_id=(credit_to,),
                                device_id_type=_MESH)

        def make_desc(li, acc, rcv, send_to):
            return pltpu.make_async_remote_copy(
                src_ref=acc, dst_ref=rcv,
                send_sem=send_sems.at[li], recv_sem=recv_sems.at[li],
                device_id=(send_to,), device_id_type=_MESH)

        descs = {}
        for li, acc, rcv, send_to, _, col0, dr in lanes:
            acc[...] = partial_chunk(_mesh_from_ring(my_p - dr), col0
                                     ).astype(jnp.bfloat16)
            pl.semaphore_wait(credits.at[li], 1)
            descs[li] = make_desc(li, acc, rcv, send_to)
            descs[li].start()

        amax_parts = []
        for s in range(N_DEV - 1):
            last = s == N_DEV - 2
            for li, acc, rcv, send_to, credit_to, col0, dr in lanes:
                if last:
                    p = partial_chunk(my_m, col0)
                else:
                    p = partial_chunk(_mesh_from_ring(my_p - dr * (2 + s)),
                                      col0)
                d = descs[li]
                d.wait_recv()
                d.wait_send()
                if last:
                    y = jnp.maximum(rcv[...].astype(jnp.float32) + p, 0.0)
                    out_ref[:, col0:col0 + NQ] = y
                    amax_parts.append(jnp.max(y))
                else:
                    acc[...] = (rcv[...].astype(jnp.float32) + p
                                ).astype(jnp.bfloat16)
                    pl.semaphore_signal(credits.at[li], inc=1,
                                        device_id=(credit_to,),
                                        device_id_type=_MESH)
                    pl.semaphore_wait(credits.at[li], 1)
                    descs[li] = make_desc(li, acc, rcv, send_to)
                    descs[li].start()

        local_amax = amax_parts[0]
        for a in amax_parts[1:]:
            local_amax = jnp.maximum(local_amax, a)

        amax_src[...] = jnp.broadcast_to(local_amax, (1, 128)
                                         ).astype(jnp.float32)
        adescs = []
        for r in range(1, N_DEV):
            d = pltpu.make_async_remote_copy(
                src_ref=amax_src, dst_ref=amax_buf.at[pl.ds(r, 1)],
                send_sem=amax_send.at[r], recv_sem=amax_recv.at[r],
                device_id=(_mesh_from_ring(my_p + r),),
                device_id_type=_MESH)
            d.start()
            adescs.append(d)
        for d in adescs:
            d.wait_send()
            d.wait_recv()
        gmax = jnp.maximum(jnp.max(amax_buf[pl.ds(1, N_DEV - 1), 0]),
                           local_amax)

        scale = gmax / 127.0
        inv = 127.0 / gmax
        out_ref[:, :NH] = jnp.clip(jnp.round(out_ref[:, :NH] * inv),
                                   0.0, 127.0) * scale
        out_ref[:, NH:] = jnp.clip(jnp.round(out_ref[:, NH:] * inv),
                                   0.0, 127.0) * scale

    return pl.pallas_call(
        body,
        out_shape=jax.ShapeDtypeStruct((MC, N), jnp.float32),
        in_specs=[pl.BlockSpec(memory_space=pltpu.VMEM),
                  pl.BlockSpec(memory_space=pltpu.VMEM)],
        out_specs=pl.BlockSpec(memory_space=pltpu.VMEM),
        scratch_shapes=[
            pltpu.VMEM((MC, NQ), jnp.bfloat16),
            pltpu.VMEM((MC, NQ), jnp.bfloat16),
            pltpu.VMEM((MC, NQ), jnp.bfloat16),
            pltpu.VMEM((MC, NQ), jnp.bfloat16),
            pltpu.VMEM((MC, NQ), jnp.bfloat16),
            pltpu.VMEM((MC, NQ), jnp.bfloat16),
            pltpu.VMEM((MC, NQ), jnp.bfloat16),
            pltpu.VMEM((MC, NQ), jnp.bfloat16),
            pltpu.SemaphoreType.DMA((4,)),
            pltpu.SemaphoreType.DMA((4,)),
            pltpu.SemaphoreType.REGULAR((4,)),
            pltpu.VMEM((1, 128), jnp.float32),
            pltpu.VMEM((N_DEV, 128), jnp.float32),
            pltpu.SemaphoreType.DMA((N_DEV,)),
            pltpu.SemaphoreType.DMA((N_DEV,)),
        ],
        compiler_params=pltpu.CompilerParams(collective_id=0),
    )(x, w_mat)
